# baseline (speedup 1.0000x reference)
import functools

import jax
import jax.numpy as jnp
from jax.experimental import pallas as pl
from jax.experimental.pallas import tpu as pltpu

_BINS = 30
_ROWS_PER_BLOCK = 64


def _ghm_kernel(p_ref, t_ref, out_ref, accs_ref, accn_ref, accd_ref, *,
                n_classes, n_steps):
    i = pl.program_id(0)

    @pl.when(i == 0)
    def _init():
        accs_ref[...] = jnp.zeros_like(accs_ref)
        accn_ref[...] = jnp.zeros_like(accn_ref)
        accd_ref[...] = jnp.zeros_like(accd_ref)

    p0 = p_ref[0, 0]
    t = t_ref[0]

    m = p0
    for c in range(1, n_classes):
        m = jnp.maximum(m, p_ref[0, c])

    denom = jnp.exp(p0 - m)
    pt = jnp.where(t == 0, p0, 0.0)
    for c in range(1, n_classes):
        pc = p_ref[0, c]
        denom = denom + jnp.exp(pc - m)
        pt = pt + jnp.where(t == c, pc, 0.0)

    shifted_t = pt - m
    ce = jnp.log(denom) - shifted_t
    py = jnp.exp(shifted_t) / denom
    g = jnp.abs(py - 1.0)

    kcf = jnp.clip(jnp.floor(g * 30.0), 0.0, _BINS - 1.0)
    e_lo = kcf / 30.0
    e_hi = (kcf + 1.0) / 30.0
    wb = kcf - (g <= e_lo).astype(jnp.float32) + (g > e_hi).astype(jnp.float32)
    r30 = jnp.float32(1.0) / jnp.float32(30.0)
    eh_lo = kcf * r30
    eh_hi = (kcf + 1.0) * r30
    hb = kcf - (g < eh_lo).astype(jnp.float32) + (g >= eh_hi).astype(jnp.float32)
    hb = jnp.minimum(hb, _BINS - 1.0)

    ones = jnp.ones_like(ce)
    for b in range(_BINS):
        mw = wb == float(b)
        s_part = jnp.where(mw, ce, 0.0).reshape(8, 8, -1).sum(axis=1)
        n_part = jnp.where(mw, ones, 0.0).reshape(8, 8, -1).sum(axis=1)
        accs_ref[8 * b:8 * (b + 1), :] += s_part
        accn_ref[8 * b:8 * (b + 1), :] += n_part

    diff = hb != wb
    ndiff = jnp.sum(diff.astype(jnp.float32))

    @pl.when(ndiff > 0.0)
    def _correct():
        for b in range(_BINS):
            dh = jnp.where(diff & (hb == float(b)), 1.0, 0.0)
            dw = jnp.where(diff & (wb == float(b)), 1.0, 0.0)
            accd_ref[b, :] += jnp.sum(dh - dw, axis=0)

    @pl.when(i == n_steps - 1)
    def _fin():
        num = jnp.float32(0.0)
        den = jnp.float32(0.0)
        for b in range(_BINS):
            s_b = jnp.sum(accs_ref[8 * b:8 * (b + 1), :])
            n_b = jnp.sum(accn_ref[8 * b:8 * (b + 1), :])
            c_b = n_b + jnp.sum(accd_ref[b, :])
            valid = n_b > 0.0
            num += jnp.where(valid, s_b / c_b, 0.0)
            den += jnp.where(valid, n_b / c_b, 0.0)
        out_ref[...] = jnp.full(out_ref.shape, num / (den + 1e-7), jnp.float32)


def kernel(preds, targets):
    b, n_classes, h, w = preds.shape
    hb = _ROWS_PER_BLOCK
    steps_per_image = h // hb
    n_steps = b * steps_per_image

    out = pl.pallas_call(
        functools.partial(_ghm_kernel, n_classes=n_classes, n_steps=n_steps),
        grid=(n_steps,),
        in_specs=[
            pl.BlockSpec((1, n_classes, hb, w),
                         lambda i: (i // steps_per_image, 0, i % steps_per_image, 0)),
            pl.BlockSpec((1, hb, w),
                         lambda i: (i // steps_per_image, i % steps_per_image, 0)),
        ],
        out_specs=pl.BlockSpec((8, 128), lambda i: (0, 0)),
        out_shape=jax.ShapeDtypeStruct((8, 128), jnp.float32),
        scratch_shapes=[
            pltpu.VMEM((8 * _BINS, w), jnp.float32),
            pltpu.VMEM((8 * _BINS, w), jnp.float32),
            pltpu.VMEM((32, w), jnp.float32),
        ],
    )(preds, targets)
    return out[0, 0]


# speedup vs baseline: 1.8293x; 1.8293x over previous
import functools

import jax
import jax.numpy as jnp
from jax.experimental import pallas as pl
from jax.experimental.pallas import tpu as pltpu

_BINS = 30
_ROWS_PER_BLOCK = 64


def _ghm_kernel(p_ref, t_ref, out_ref, accs_ref, accn_ref, accd_ref, *,
                n_classes, n_steps):
    i = pl.program_id(0)

    @pl.when(i == 0)
    def _init():
        accs_ref[...] = jnp.zeros_like(accs_ref)
        accn_ref[...] = jnp.zeros_like(accn_ref)
        accd_ref[...] = jnp.zeros_like(accd_ref)

    p0 = p_ref[0, 0]
    t = t_ref[0]

    m = p0
    for c in range(1, n_classes):
        m = jnp.maximum(m, p_ref[0, c])

    denom = jnp.exp(p0 - m)
    pt = jnp.where(t == 0, p0, 0.0)
    for c in range(1, n_classes):
        pc = p_ref[0, c]
        denom = denom + jnp.exp(pc - m)
        pt = pt + jnp.where(t == c, pc, 0.0)

    shifted_t = pt - m
    ce = jnp.log(denom) - shifted_t
    py = jnp.exp(shifted_t) / denom
    g = jnp.abs(py - 1.0)

    kcf = jnp.clip(jnp.floor(g * 30.0), 0.0, _BINS - 1.0)
    e_lo = kcf / 30.0
    e_hi = (kcf + 1.0) / 30.0
    wb = kcf - (g <= e_lo).astype(jnp.float32) + (g > e_hi).astype(jnp.float32)
    r30 = jnp.float32(1.0) / jnp.float32(30.0)
    eh_lo = kcf * r30
    eh_hi = (kcf + 1.0) * r30
    hb = kcf - (g < eh_lo).astype(jnp.float32) + (g >= eh_hi).astype(jnp.float32)
    hb = jnp.minimum(hb, _BINS - 1.0)

    def _rowsum8(x):
        a = x[0:32, :] + x[32:64, :]
        a = a[0:16, :] + a[16:32, :]
        return a[0:8, :] + a[8:16, :]

    ones = jnp.ones_like(ce)
    for b in range(_BINS):
        mw = wb == float(b)
        accs_ref[8 * b:8 * (b + 1), :] += _rowsum8(jnp.where(mw, ce, 0.0))
        accn_ref[8 * b:8 * (b + 1), :] += _rowsum8(jnp.where(mw, ones, 0.0))

    diff = hb != wb
    ndiff = jnp.sum(diff.astype(jnp.float32))

    @pl.when(ndiff > 0.0)
    def _correct():
        for b in range(_BINS):
            dh = jnp.where(diff & (hb == float(b)), 1.0, 0.0)
            dw = jnp.where(diff & (wb == float(b)), 1.0, 0.0)
            accd_ref[b, :] += jnp.sum(dh - dw, axis=0)

    @pl.when(i == n_steps - 1)
    def _fin():
        num = jnp.float32(0.0)
        den = jnp.float32(0.0)
        for b in range(_BINS):
            s_b = jnp.sum(accs_ref[8 * b:8 * (b + 1), :])
            n_b = jnp.sum(accn_ref[8 * b:8 * (b + 1), :])
            c_b = n_b + jnp.sum(accd_ref[b, :])
            valid = n_b > 0.0
            num += jnp.where(valid, s_b / c_b, 0.0)
            den += jnp.where(valid, n_b / c_b, 0.0)
        out_ref[...] = jnp.full(out_ref.shape, num / (den + 1e-7), jnp.float32)


def kernel(preds, targets):
    b, n_classes, h, w = preds.shape
    hb = _ROWS_PER_BLOCK
    steps_per_image = h // hb
    n_steps = b * steps_per_image

    out = pl.pallas_call(
        functools.partial(_ghm_kernel, n_classes=n_classes, n_steps=n_steps),
        grid=(n_steps,),
        in_specs=[
            pl.BlockSpec((1, n_classes, hb, w),
                         lambda i: (i // steps_per_image, 0, i % steps_per_image, 0)),
            pl.BlockSpec((1, hb, w),
                         lambda i: (i // steps_per_image, i % steps_per_image, 0)),
        ],
        out_specs=pl.BlockSpec((8, 128), lambda i: (0, 0)),
        out_shape=jax.ShapeDtypeStruct((8, 128), jnp.float32),
        scratch_shapes=[
            pltpu.VMEM((8 * _BINS, w), jnp.float32),
            pltpu.VMEM((8 * _BINS, w), jnp.float32),
            pltpu.VMEM((32, w), jnp.float32),
        ],
    )(preds, targets)
    return out[0, 0]
